# baseline (speedup 1.0000x reference)
import jax
import jax.numpy as jnp
from jax import lax
from jax.experimental import pallas as pl
from jax.experimental.pallas import tpu as pltpu

_N, _C, _H, _W = 128, 64, 56, 56
_NSL = _N * _C
_NP = (_H // 2) * (_W // 2)
_Q = 4 * _NP
_L = 28 * 56
_SBLK = 128

_KS0 = 0
_KS1 = 42
_KS2 = _KS0 ^ _KS1 ^ 0x1BD11BDA


def _rotl(x, r):
    return (x << jnp.uint32(r)) | (x >> jnp.uint32(32 - r))


def _threefry_bits(t):
    ks = (jnp.uint32(_KS0), jnp.uint32(_KS1), jnp.uint32(_KS2))
    rot = ((13, 15, 26, 6), (17, 29, 16, 24))
    x1 = t + ks[1]
    x0 = x1
    x1 = _rotl(x1, 13) ^ x1
    for r in (15, 26, 6):
        x0 = x0 + x1
        x1 = _rotl(x1, r)
        x1 = x1 ^ x0
    x0 = x0 + ks[1]
    x1 = x1 + ks[2] + jnp.uint32(1)
    for i in range(1, 5):
        for r in rot[i % 2]:
            x0 = x0 + x1
            x1 = _rotl(x1, r)
            x1 = x1 ^ x0
        x0 = x0 + ks[(i + 1) % 3]
        x1 = x1 + ks[(i + 2) % 3] + jnp.uint32(i + 1)
    return x0 ^ x1


def _keys(t):
    return lax.bitcast_convert_type(
        _threefry_bits(lax.bitcast_convert_type(t, jnp.uint32)) >> jnp.uint32(9),
        jnp.int32,
    )


def _shuffle_kernel(qmap_ref, v_ref, o_ref):
    s0 = pl.program_id(0) * _SBLK
    row = lax.broadcasted_iota(jnp.int32, (_SBLK, _L), 0)
    lane = lax.broadcasted_iota(jnp.int32, (_SBLK, _L), 1)
    srow = (s0 + row) * _Q
    qmap = qmap_ref[0]

    u1 = _keys(srow + qmap)
    u3 = _keys(srow + qmap + 2)

    d0 = (lane & 1) == 0
    u2 = jnp.where(d0, pltpu.roll(u1, _L - 1, 1), pltpu.roll(u1, 1, 1))
    u4 = jnp.where(d0, pltpu.roll(u3, _L - 1, 1), pltpu.roll(u3, 1, 1))

    c12 = (u1 < u2) | ((u1 == u2) & d0)
    c34 = (u3 < u4) | ((u3 == u4) & d0)
    c13 = u1 <= u3
    c14 = u1 <= u4
    c23 = u2 <= u3
    c24 = u2 <= u4
    i32 = lambda b: b.astype(jnp.int32)
    r1 = 3 - i32(c12) - i32(c13) - i32(c14)
    r2 = 2 + i32(c12) - i32(c23) - i32(c24)
    r3 = i32(c13) + i32(c23) + 1 - i32(c34)
    r4 = i32(c14) + i32(c24) + i32(c34)

    v1, v2, v3, v4 = (v_ref[k : k + 1, :] for k in range(4))

    jout_e = lane & 1
    jout_o = jout_e + 2
    out_e = jnp.where(
        r1 == jout_e,
        v1,
        jnp.where(r2 == jout_e, v2, jnp.where(r3 == jout_e, v3, v4)),
    )
    out_o = jnp.where(
        r1 == jout_o,
        v1,
        jnp.where(r2 == jout_o, v2, jnp.where(r3 == jout_o, v3, v4)),
    )

    o_ref[:, pl.Slice(0, 28, 2), :] = out_e.reshape(_SBLK, 28, _W)
    o_ref[:, pl.Slice(1, 28, 2), :] = out_o.reshape(_SBLK, 28, _W)


def kernel(T):
    n, c, h, w = T.shape

    l = jnp.arange(_L)
    a, wl = l // 56, l % 56
    b, d = wl // 2, wl % 2
    p = 28 * a + b
    qmap = (4 * p + d).astype(jnp.int32).reshape(1, _L)

    vals = T[0, 0].reshape(28, 2, 28, 2).transpose(1, 3, 0, 2).reshape(4, _NP)
    k1 = d
    k2 = d ^ 1
    k3 = 2 + d
    k4 = 2 + (d ^ 1)
    vquad = jnp.stack(
        [vals[k1, p], vals[k2, p], vals[k3, p], vals[k4, p]], axis=0
    )

    out = pl.pallas_call(
        _shuffle_kernel,
        out_shape=jax.ShapeDtypeStruct((_NSL, _H, _W), jnp.float32),
        grid=(_NSL // _SBLK,),
        in_specs=[
            pl.BlockSpec((1, _L), lambda i: (0, 0)),
            pl.BlockSpec((4, _L), lambda i: (0, 0)),
        ],
        out_specs=pl.BlockSpec((_SBLK, _H, _W), lambda i: (i, 0, 0)),
    )(qmap, vquad)

    return out


# speedup vs baseline: 1.5785x; 1.5785x over previous
import jax
import jax.numpy as jnp
from jax import lax
from jax.experimental import pallas as pl

_N, _C, _H, _W = 128, 64, 56, 56
_HP, _WP = _H // 2, _W // 2
_NP = _HP * _WP
_Q = 4 * _NP
_CBLK = 4

_KS0 = 0
_KS1 = 42
_KS2 = _KS0 ^ _KS1 ^ 0x1BD11BDA


def _rotl(x, r):
    return (x << jnp.uint32(r)) | (x >> jnp.uint32(32 - r))


def _threefry_bits(t):
    ks = (jnp.uint32(_KS0), jnp.uint32(_KS1), jnp.uint32(_KS2))
    rot = ((13, 15, 26, 6), (17, 29, 16, 24))
    x1 = t + ks[1]
    x0 = x1
    x1 = _rotl(x1, 13) ^ x1
    for r in (15, 26, 6):
        x0 = x0 + x1
        x1 = _rotl(x1, r)
        x1 = x1 ^ x0
    x0 = x0 + ks[1]
    x1 = x1 + ks[2] + jnp.uint32(1)
    for i in range(1, 5):
        for r in rot[i % 2]:
            x0 = x0 + x1
            x1 = _rotl(x1, r)
            x1 = x1 ^ x0
        x0 = x0 + ks[(i + 1) % 3]
        x1 = x1 + ks[(i + 2) % 3] + jnp.uint32(i + 1)
    return x0 ^ x1


def _keys(t):
    return lax.bitcast_convert_type(
        _threefry_bits(lax.bitcast_convert_type(t, jnp.uint32)) >> jnp.uint32(9),
        jnp.int32,
    )


def _shuffle_kernel(v_ref, o_ref):
    c0 = pl.program_id(0) * _CBLK
    shp = (_CBLK, _HP, _WP, _N)
    ci = lax.broadcasted_iota(jnp.int32, shp, 0)
    ai = lax.broadcasted_iota(jnp.int32, shp, 1)
    bi = lax.broadcasted_iota(jnp.int32, shp, 2)
    ni = lax.broadcasted_iota(jnp.int32, shp, 3)
    base = ni * (_C * _Q) + (c0 + ci) * _Q + 112 * ai + 4 * bi

    u = [_keys(base + j) for j in range(4)]

    b01 = (u[0] <= u[1]).astype(jnp.int32)
    b02 = (u[0] <= u[2]).astype(jnp.int32)
    b03 = (u[0] <= u[3]).astype(jnp.int32)
    b12 = (u[1] <= u[2]).astype(jnp.int32)
    b13 = (u[1] <= u[3]).astype(jnp.int32)
    b23 = (u[2] <= u[3]).astype(jnp.int32)
    rank = (
        3 - b01 - b02 - b03,
        2 + b01 - b12 - b13,
        1 + b02 + b12 - b23,
        b03 + b13 + b23,
    )

    v = [v_ref[k] for k in range(4)]
    for j in range(4):
        o = jnp.where(
            rank[0] == j,
            v[0],
            jnp.where(rank[1] == j, v[1], jnp.where(rank[2] == j, v[2], v[3])),
        )
        dh, dw = j // 2, j % 2
        o_ref[:, pl.Slice(dh, _HP, 2), pl.Slice(dw, _WP, 2), :] = o


def kernel(T):
    n, c, h, w = T.shape
    vals = T[0, 0].reshape(_HP, 2, _WP, 2).transpose(1, 3, 0, 2)
    vals4 = jnp.broadcast_to(vals.reshape(4, _HP, _WP, 1), (4, _HP, _WP, _N))

    out = pl.pallas_call(
        _shuffle_kernel,
        out_shape=jax.ShapeDtypeStruct((_C, _H, _W, _N), jnp.float32),
        grid=(_C // _CBLK,),
        in_specs=[pl.BlockSpec((4, _HP, _WP, _N), lambda i: (0, 0, 0, 0))],
        out_specs=pl.BlockSpec((_CBLK, _H, _W, _N), lambda i: (i, 0, 0, 0)),
    )(vals4)

    return out.transpose(3, 0, 1, 2)
